# Initial kernel scaffold; baseline (speedup 1.0000x reference)
import jax
import jax.numpy as jnp
from jax.experimental import pallas as pl


def kernel(context_images, context_labels, target_logits, val_freqs):
    raise NotImplementedError("write your pallas kernel here")


# speedup vs baseline: 673.1597x; 673.1597x over previous
import functools

import jax
import jax.numpy as jnp
import numpy as np
from jax.experimental import pallas as pl

_C = 150
_BINS = 15
_HW = 512 * 512
_BLK = 4096

_WIDTH = np.float32(1.0) / np.float32(_BINS)


def _body(x_ref, vf_ref, o_ref):
    x = x_ref[...]
    m = jnp.max(x, axis=0, keepdims=True)
    e = jnp.exp(x - m)
    s = jnp.sum(e, axis=0, keepdims=True)
    p = e / s
    b = jnp.clip(jnp.floor(p / _WIDTH), 0.0, float(_BINS - 1))
    vf = vf_ref[...]
    cal = jnp.zeros_like(x)
    for k in range(_BINS):
        cal = jnp.where(b == float(k), vf[:, k : k + 1], cal)
    ssum = jnp.sum(cal, axis=0, keepdims=True)
    ssum = jnp.where(ssum == 0.0, 1.0, ssum)
    o_ref[...] = cal / ssum


@jax.jit
def _run(logits2d, val_freqs):
    grid = _HW // _BLK
    return pl.pallas_call(
        _body,
        grid=(grid,),
        in_specs=[
            pl.BlockSpec((_C, _BLK), lambda i: (0, i)),
            pl.BlockSpec((_C, _BINS), lambda i: (0, 0)),
        ],
        out_specs=pl.BlockSpec((_C, _BLK), lambda i: (0, i)),
        out_shape=jax.ShapeDtypeStruct((_C, _HW), jnp.float32),
    )(logits2d, val_freqs)


def kernel(context_images, context_labels, target_logits, val_freqs):
    lg = target_logits.reshape(_C, _HW)
    out = _run(lg, val_freqs)
    return out.reshape(1, _C, 512, 512)
